# Initial kernel scaffold; baseline (speedup 1.0000x reference)
import jax
import jax.numpy as jnp
from jax.experimental import pallas as pl


def kernel(x, edge_index, W, Wb, a, ab):
    raise NotImplementedError("write your pallas kernel here")


# speedup vs baseline: 34.0257x; 34.0257x over previous
import jax
import jax.numpy as jnp
from jax import lax
from jax.experimental import pallas as pl
from jax.experimental.pallas import tpu as pltpu
from jax.experimental.pallas import tpu_sc as plsc

N = 10000
E = 320000
F = 128
H = 8
D = 16
ALPHA = 0.2

NC = 2
NS = 16
NP = 10240
RPT = NP // NS
EPW = E // (NC * NS)
B = 80
NCHUNK = EPW // B


def _tc_pre(x_ref, wct_ref, wb_ref, aall_ref, ab_ref,
            wh_ref, tall_ref, bmax_ref):
    i = pl.program_id(0)
    wh = jnp.dot(x_ref[...], wct_ref[...], preferred_element_type=jnp.float32)
    wh = wh + wb_ref[...]
    wh_ref[...] = wh
    tall = jnp.dot(wh, aall_ref[...], preferred_element_type=jnp.float32)
    tall = tall + ab_ref[...]
    tall_ref[...] = tall
    m = jnp.max(tall, axis=0, keepdims=True)

    @pl.when(i == 0)
    def _():
        bmax_ref[...] = m

    @pl.when(i != 0)
    def _():
        bmax_ref[...] = jnp.maximum(bmax_ref[...], m)


def _tc_post(a0_ref, a1_ref, d0_ref, d1_ref, emat_ref, out_ref):
    dsum = d0_ref[...] + d1_ref[...]
    dx = jnp.dot(dsum, emat_ref[...], preferred_element_type=jnp.float32)
    acc = a0_ref[...] + a1_ref[...]
    out_ref[...] = acc / jnp.maximum(dx, 1e-30)


def _sc_edge(src_hbm, dst_hbm, wh_hbm, tall_hbm, bnd_hbm,
             acc_out, ex_out,
             srcv, dstv, gv, rowsv, exbv, bndv,
             acc_sh):
    cid = lax.axis_index("c")
    sid = lax.axis_index("s")
    wid = cid * NS + sid
    sync_copy = pltpu.sync_copy
    z16 = jnp.zeros((16,), jnp.float32)
    iota16 = lax.iota(jnp.int32, 16)
    r0 = sid * RPT

    @pl.loop(0, B)
    def _(i):
        for h in range(F // 16):
            rowsv[i, pl.ds(h * 16, 16)] = z16

    sync_copy(bnd_hbm, bndv)

    @pl.loop(0, RPT // B)
    def _(k):
        base_r = r0 + k * B

        @pl.loop(0, B // 16)
        def _(j):
            srcv[pl.ds(j * 16, 16)] = iota16 + (base_r + j * 16)

        sync_copy(rowsv, acc_sh.at[srcv])

    plsc.subcore_barrier()

    @pl.loop(0, NCHUNK)
    def _(c):
        base = wid * EPW + c * B
        sync_copy(src_hbm.at[pl.ds(base, B)], srcv)
        sync_copy(dst_hbm.at[pl.ds(base, B)], dstv)
        sync_copy(tall_hbm.at[srcv], gv)

        @pl.loop(0, B)
        def _(i):
            exbv[i, :] = gv[i, pl.ds(0, 16)]

        sync_copy(tall_hbm.at[dstv], gv)
        bnd = bndv[...]

        @pl.loop(0, B)
        def _(i):
            z = exbv[i, :] + gv[i, pl.ds(16, 16)]
            z = jnp.maximum(z, ALPHA * z)
            exbv[i, :] = jnp.exp(z - bnd)

        sync_copy(wh_hbm.at[srcv], rowsv)

        @pl.loop(0, B)
        def _(i):
            ex = exbv[i, :]
            for h in range(H):
                s = ex[h]
                sl = pl.ds(h * 16, 16)
                rowsv[i, sl] = rowsv[i, sl] * s

        sync_copy(exbv, ex_out.at[pl.ds(base, B)])
        sync_copy(rowsv, acc_sh.at[dstv], add=True)

    plsc.subcore_barrier()

    @pl.loop(0, RPT // B)
    def _(k):
        base_r = r0 + k * B

        @pl.loop(0, B // 16)
        def _(j):
            srcv[pl.ds(j * 16, 16)] = iota16 + (base_r + j * 16)

        sync_copy(acc_sh.at[srcv], rowsv)
        sync_copy(rowsv, acc_out.at[cid, pl.ds(base_r, B)])


def _sc_den(dst_hbm, ex_hbm, den_out, dstv, exv, payv, den_sh):
    cid = lax.axis_index("c")
    sid = lax.axis_index("s")
    wid = cid * NS + sid
    sync_copy = pltpu.sync_copy
    z16 = jnp.zeros((16,), jnp.float32)
    iota16 = lax.iota(jnp.int32, 16)
    r0 = sid * RPT

    @pl.loop(0, B)
    def _(i):
        for h in range(F // 16):
            payv[i, pl.ds(h * 16, 16)] = z16

    @pl.loop(0, RPT // B)
    def _(k):
        base_r = r0 + k * B

        @pl.loop(0, B // 16)
        def _(j):
            dstv[pl.ds(j * 16, 16)] = iota16 + (base_r + j * 16)

        sync_copy(payv, den_sh.at[dstv])

    plsc.subcore_barrier()

    @pl.loop(0, NCHUNK)
    def _(c):
        base = wid * EPW + c * B
        sync_copy(dst_hbm.at[pl.ds(base, B)], dstv)
        sync_copy(ex_hbm.at[pl.ds(base, B)], exv)

        @pl.loop(0, B)
        def _(i):
            payv[i, pl.ds(0, 16)] = exv[i, :]

        sync_copy(payv, den_sh.at[dstv], add=True)

    plsc.subcore_barrier()

    @pl.loop(0, RPT // B)
    def _(k):
        base_r = r0 + k * B

        @pl.loop(0, B // 16)
        def _(j):
            dstv[pl.ds(j * 16, 16)] = iota16 + (base_r + j * 16)

        sync_copy(den_sh.at[dstv], payv)
        sync_copy(payv, den_out.at[cid, pl.ds(base_r, B)])


def kernel(x, edge_index, W, Wb, a, ab):
    src = edge_index[0]
    dst = edge_index[1]

    wct = jnp.transpose(W, (2, 0, 1)).reshape(F, H * D)
    wb2 = Wb.reshape(1, H * D)
    a_src = a[:, :D]
    a_dst = a[:, D:]
    eye8 = jnp.eye(H, dtype=jnp.float32)
    c1 = (a_src[:, :, None] * eye8[:, None, :]).reshape(H * D, H)
    c2 = (a_dst[:, :, None] * eye8[:, None, :]).reshape(H * D, H)
    aall = jnp.concatenate(
        [c1, c1, c2, c2, jnp.zeros((H * D, 128 - 4 * H), jnp.float32)],
        axis=1)
    ab16 = jnp.concatenate([ab, ab])
    ab128 = jnp.concatenate(
        [jnp.zeros((16,), jnp.float32), ab16,
         jnp.zeros((96,), jnp.float32)]).reshape(1, 128)

    blk = 1000
    grid = (N // blk,)
    wh, tall, bmax = pl.pallas_call(
        _tc_pre,
        grid=grid,
        in_specs=[
            pl.BlockSpec((blk, F), lambda i: (i, 0)),
            pl.BlockSpec((F, H * D), lambda i: (0, 0)),
            pl.BlockSpec((1, H * D), lambda i: (0, 0)),
            pl.BlockSpec((H * D, 128), lambda i: (0, 0)),
            pl.BlockSpec((1, 128), lambda i: (0, 0)),
        ],
        out_specs=[
            pl.BlockSpec((blk, H * D), lambda i: (i, 0)),
            pl.BlockSpec((blk, 128), lambda i: (i, 0)),
            pl.BlockSpec((1, 128), lambda i: (0, 0)),
        ],
        out_shape=[
            jax.ShapeDtypeStruct((N, H * D), jnp.float32),
            jax.ShapeDtypeStruct((N, 128), jnp.float32),
            jax.ShapeDtypeStruct((1, 128), jnp.float32),
        ],
    )(x, wct, wb2, aall, ab128)

    braw = bmax[0, :D] + bmax[0, D:2 * D]
    bnd16 = jnp.maximum(braw, ALPHA * braw)

    mesh = plsc.VectorSubcoreMesh(core_axis_name="c", subcore_axis_name="s")
    sc_a = pl.kernel(
        _sc_edge,
        out_type=[
            jax.ShapeDtypeStruct((NC, NP, F), jnp.float32),
            jax.ShapeDtypeStruct((E, D), jnp.float32),
        ],
        mesh=mesh,
        scratch_types=[
            pltpu.VMEM((B,), jnp.int32),
            pltpu.VMEM((B,), jnp.int32),
            pltpu.VMEM((B, 128), jnp.float32),
            pltpu.VMEM((B, F), jnp.float32),
            pltpu.VMEM((B, 16), jnp.float32),
            pltpu.VMEM((16,), jnp.float32),
            pltpu.VMEM_SHARED((NP, F), jnp.float32),
        ],
    )
    acc_p, exr = sc_a(src, dst, wh, tall, bnd16)

    sc_b = pl.kernel(
        _sc_den,
        out_type=jax.ShapeDtypeStruct((NC, NP, F), jnp.float32),
        mesh=mesh,
        scratch_types=[
            pltpu.VMEM((B,), jnp.int32),
            pltpu.VMEM((B, 16), jnp.float32),
            pltpu.VMEM((B, F), jnp.float32),
            pltpu.VMEM_SHARED((NP, F), jnp.float32),
        ],
    )
    den_p = sc_b(dst, exr)

    emat = (jnp.eye(H, dtype=jnp.float32)[:, :, None]
            * jnp.ones((1, 1, D), jnp.float32)).reshape(H, H * D)
    emat = jnp.concatenate([emat, jnp.zeros((H, H * D), jnp.float32)], axis=0)

    out = pl.pallas_call(
        _tc_post,
        grid=grid,
        in_specs=[
            pl.BlockSpec((blk, F), lambda i: (i, 0)),
            pl.BlockSpec((blk, F), lambda i: (i, 0)),
            pl.BlockSpec((blk, D), lambda i: (i, 0)),
            pl.BlockSpec((blk, D), lambda i: (i, 0)),
            pl.BlockSpec((2 * H, H * D), lambda i: (0, 0)),
        ],
        out_specs=pl.BlockSpec((blk, H * D), lambda i: (i, 0)),
        out_shape=jax.ShapeDtypeStruct((N, H * D), jnp.float32),
    )(acc_p[0, :N], acc_p[1, :N], den_p[0, :N, :D], den_p[1, :N, :D], emat)
    return out


# speedup vs baseline: 37.9168x; 1.1144x over previous
import jax
import jax.numpy as jnp
from jax import lax
from jax.experimental import pallas as pl
from jax.experimental.pallas import tpu as pltpu
from jax.experimental.pallas import tpu_sc as plsc

N = 10000
E = 320000
F = 128
H = 8
D = 16
ALPHA = 0.2

NC = 2
NS = 16
NP = 10240
RPT = NP // NS
EPW = E // (NC * NS)
B = 80
NCHUNK = EPW // B


def _tc_pre(x_ref, wct_ref, wb_ref, aall_ref, ab_ref,
            wh_ref, tall_ref, bmax_ref):
    i = pl.program_id(0)
    wh = jnp.dot(x_ref[...], wct_ref[...], preferred_element_type=jnp.float32)
    wh = wh + wb_ref[...]
    wh_ref[...] = wh
    tall = jnp.dot(wh, aall_ref[...], preferred_element_type=jnp.float32)
    tall = tall + ab_ref[...]
    tall_ref[...] = tall
    m = jnp.max(tall, axis=0, keepdims=True)

    @pl.when(i == 0)
    def _():
        bmax_ref[...] = m

    @pl.when(i != 0)
    def _():
        bmax_ref[...] = jnp.maximum(bmax_ref[...], m)


def _tc_post(a0_ref, a1_ref, d0_ref, d1_ref, emat_ref, out_ref):
    dsum = d0_ref[...] + d1_ref[...]
    dx = jnp.dot(dsum, emat_ref[...], preferred_element_type=jnp.float32)
    acc = a0_ref[...] + a1_ref[...]
    out_ref[...] = acc / jnp.maximum(dx, 1e-30)


def _sc_edge(src_hbm, dst_hbm, wh_hbm, tall_hbm, bnd_hbm,
             acc_out, ex_out,
             srcv, dstv, gv, rowsv, exbv, bndv, dsem,
             acc_sh):
    cid = lax.axis_index("c")
    sid = lax.axis_index("s")
    wid = cid * NS + sid
    sync_copy = pltpu.sync_copy
    z16 = jnp.zeros((16,), jnp.float32)
    iota16 = lax.iota(jnp.int32, 16)
    r0 = sid * RPT

    @pl.loop(0, B)
    def _(i):
        for h in range(F // 16):
            rowsv[i, pl.ds(h * 16, 16)] = z16

    sync_copy(bnd_hbm, bndv)

    @pl.loop(0, RPT // B)
    def _(k):
        base_r = r0 + k * B

        @pl.loop(0, B // 16)
        def _(j):
            srcv[pl.ds(j * 16, 16)] = iota16 + (base_r + j * 16)

        sync_copy(rowsv, acc_sh.at[srcv])

    plsc.subcore_barrier()

    @pl.loop(0, NCHUNK)
    def _(c):
        base = wid * EPW + c * B
        sync_copy(src_hbm.at[pl.ds(base, B)], srcv)
        sync_copy(dst_hbm.at[pl.ds(base, B)], dstv)
        sync_copy(tall_hbm.at[srcv], gv)
        wh_cp = pltpu.async_copy(wh_hbm.at[srcv], rowsv, dsem)

        @pl.loop(0, B)
        def _(i):
            exbv[i, :] = gv[i, pl.ds(0, 16)]

        sync_copy(tall_hbm.at[dstv], gv)
        bnd = bndv[...]

        @pl.loop(0, B)
        def _(i):
            z = exbv[i, :] + gv[i, pl.ds(16, 16)]
            z = jnp.maximum(z, ALPHA * z)
            exbv[i, :] = jnp.exp(z - bnd)

        wh_cp.wait()

        @pl.loop(0, B)
        def _(i):
            ex = exbv[i, :]
            for h in range(H):
                s = ex[h]
                sl = pl.ds(h * 16, 16)
                rowsv[i, sl] = rowsv[i, sl] * s

        sync_copy(exbv, ex_out.at[pl.ds(base, B)])
        sync_copy(rowsv, acc_sh.at[dstv], add=True)

    plsc.subcore_barrier()

    @pl.loop(0, RPT // B)
    def _(k):
        base_r = r0 + k * B

        @pl.loop(0, B // 16)
        def _(j):
            srcv[pl.ds(j * 16, 16)] = iota16 + (base_r + j * 16)

        sync_copy(acc_sh.at[srcv], rowsv)
        sync_copy(rowsv, acc_out.at[cid, pl.ds(base_r, B)])


def _sc_den(dst_hbm, ex_hbm, den_out, dstv, exv, payv, den_sh):
    cid = lax.axis_index("c")
    sid = lax.axis_index("s")
    wid = cid * NS + sid
    sync_copy = pltpu.sync_copy
    z16 = jnp.zeros((16,), jnp.float32)
    iota16 = lax.iota(jnp.int32, 16)
    r0 = sid * RPT

    @pl.loop(0, B)
    def _(i):
        for h in range(F // 16):
            payv[i, pl.ds(h * 16, 16)] = z16

    @pl.loop(0, RPT // B)
    def _(k):
        base_r = r0 + k * B

        @pl.loop(0, B // 16)
        def _(j):
            dstv[pl.ds(j * 16, 16)] = iota16 + (base_r + j * 16)

        sync_copy(payv, den_sh.at[dstv])

    plsc.subcore_barrier()

    @pl.loop(0, NCHUNK)
    def _(c):
        base = wid * EPW + c * B
        sync_copy(dst_hbm.at[pl.ds(base, B)], dstv)
        sync_copy(ex_hbm.at[pl.ds(base, B)], exv)

        @pl.loop(0, B)
        def _(i):
            payv[i, pl.ds(0, 16)] = exv[i, :]

        sync_copy(payv, den_sh.at[dstv], add=True)

    plsc.subcore_barrier()

    @pl.loop(0, RPT // B)
    def _(k):
        base_r = r0 + k * B

        @pl.loop(0, B // 16)
        def _(j):
            dstv[pl.ds(j * 16, 16)] = iota16 + (base_r + j * 16)

        sync_copy(den_sh.at[dstv], payv)
        sync_copy(payv, den_out.at[cid, pl.ds(base_r, B)])


def kernel(x, edge_index, W, Wb, a, ab):
    src = edge_index[0]
    dst = edge_index[1]

    wct = jnp.transpose(W, (2, 0, 1)).reshape(F, H * D)
    wb2 = Wb.reshape(1, H * D)
    a_src = a[:, :D]
    a_dst = a[:, D:]
    eye8 = jnp.eye(H, dtype=jnp.float32)
    c1 = (a_src[:, :, None] * eye8[:, None, :]).reshape(H * D, H)
    c2 = (a_dst[:, :, None] * eye8[:, None, :]).reshape(H * D, H)
    aall = jnp.concatenate(
        [c1, c1, c2, c2, jnp.zeros((H * D, 128 - 4 * H), jnp.float32)],
        axis=1)
    ab16 = jnp.concatenate([ab, ab])
    ab128 = jnp.concatenate(
        [jnp.zeros((16,), jnp.float32), ab16,
         jnp.zeros((96,), jnp.float32)]).reshape(1, 128)

    blk = 1000
    grid = (N // blk,)
    wh, tall, bmax = pl.pallas_call(
        _tc_pre,
        grid=grid,
        in_specs=[
            pl.BlockSpec((blk, F), lambda i: (i, 0)),
            pl.BlockSpec((F, H * D), lambda i: (0, 0)),
            pl.BlockSpec((1, H * D), lambda i: (0, 0)),
            pl.BlockSpec((H * D, 128), lambda i: (0, 0)),
            pl.BlockSpec((1, 128), lambda i: (0, 0)),
        ],
        out_specs=[
            pl.BlockSpec((blk, H * D), lambda i: (i, 0)),
            pl.BlockSpec((blk, 128), lambda i: (i, 0)),
            pl.BlockSpec((1, 128), lambda i: (0, 0)),
        ],
        out_shape=[
            jax.ShapeDtypeStruct((N, H * D), jnp.float32),
            jax.ShapeDtypeStruct((N, 128), jnp.float32),
            jax.ShapeDtypeStruct((1, 128), jnp.float32),
        ],
    )(x, wct, wb2, aall, ab128)

    braw = bmax[0, :D] + bmax[0, D:2 * D]
    bnd16 = jnp.maximum(braw, ALPHA * braw)

    mesh = plsc.VectorSubcoreMesh(core_axis_name="c", subcore_axis_name="s")
    sc_a = pl.kernel(
        _sc_edge,
        out_type=[
            jax.ShapeDtypeStruct((NC, NP, F), jnp.float32),
            jax.ShapeDtypeStruct((E, D), jnp.float32),
        ],
        mesh=mesh,
        scratch_types=[
            pltpu.VMEM((B,), jnp.int32),
            pltpu.VMEM((B,), jnp.int32),
            pltpu.VMEM((B, 128), jnp.float32),
            pltpu.VMEM((B, F), jnp.float32),
            pltpu.VMEM((B, 16), jnp.float32),
            pltpu.VMEM((16,), jnp.float32),
            pltpu.SemaphoreType.DMA,
            pltpu.VMEM_SHARED((NP, F), jnp.float32),
        ],
    )
    acc_p, exr = sc_a(src, dst, wh, tall, bnd16)

    sc_b = pl.kernel(
        _sc_den,
        out_type=jax.ShapeDtypeStruct((NC, NP, F), jnp.float32),
        mesh=mesh,
        scratch_types=[
            pltpu.VMEM((B,), jnp.int32),
            pltpu.VMEM((B, 16), jnp.float32),
            pltpu.VMEM((B, F), jnp.float32),
            pltpu.VMEM_SHARED((NP, F), jnp.float32),
        ],
    )
    den_p = sc_b(dst, exr)

    emat = (jnp.eye(H, dtype=jnp.float32)[:, :, None]
            * jnp.ones((1, 1, D), jnp.float32)).reshape(H, H * D)
    emat = jnp.concatenate([emat, jnp.zeros((H, H * D), jnp.float32)], axis=0)

    out = pl.pallas_call(
        _tc_post,
        grid=grid,
        in_specs=[
            pl.BlockSpec((blk, F), lambda i: (i, 0)),
            pl.BlockSpec((blk, F), lambda i: (i, 0)),
            pl.BlockSpec((blk, D), lambda i: (i, 0)),
            pl.BlockSpec((blk, D), lambda i: (i, 0)),
            pl.BlockSpec((2 * H, H * D), lambda i: (0, 0)),
        ],
        out_specs=pl.BlockSpec((blk, H * D), lambda i: (i, 0)),
        out_shape=jax.ShapeDtypeStruct((N, H * D), jnp.float32),
    )(acc_p[0, :N], acc_p[1, :N], den_p[0, :N, :D], den_p[1, :N, :D], emat)
    return out
